# baseline (speedup 1.0000x reference)
import functools

import jax
import jax.numpy as jnp
import numpy as np
from jax.experimental import pallas as pl
from jax.experimental.pallas import tpu as pltpu
from jax.experimental.pallas import tpu_sc as plsc

H = 128
L = 3
GH = 256
B = 8
T = 12
N = 325
E = 2600

NPAD = 352
EPAD2 = 2688
TPAD = 16
P = 336
NROWS = B * N
NRPAD = 2688


def _lstm_body(x_ref, m0_ref, wh_ref, wi_ref, b_ref, out_ref, xsa, xsb,
               gihf_s, gihb_s, gih0f_s, gih0b_s):
    f32 = jnp.float32
    bf16 = jnp.bfloat16

    def sg(x):
        return 0.5 * jnp.tanh(0.5 * x) + 0.5

    def cell(g, c):
        ig = sg(g[:, :H])
        fg = sg(g[:, H:2 * H])
        gg = jnp.tanh(g[:, 2 * H:3 * H])
        og = sg(g[:, 3 * H:])
        c2 = fg * c + ig * gg
        h2 = og * jnp.tanh(c2)
        return h2, c2

    def run_layer(l, xs_in, xs_out):
        whf = wh_ref[2 * l]
        whb = wh_ref[2 * l + 1]
        z = jnp.zeros((P, H), f32)

        if l == 0:
            xb = x_ref[...].astype(bf16)
            gih0f_s[...] = jnp.dot(xb, m0_ref[0], preferred_element_type=f32)
            gih0b_s[...] = jnp.dot(xb, m0_ref[1], preferred_element_type=f32)

            def gih(d, t):
                src = gih0f_s if d == 0 else gih0b_s
                return src[:, t * 4 * H:(t + 1) * 4 * H]
        else:
            wif = wi_ref[2 * (l - 1)]
            wib = wi_ref[2 * (l - 1) + 1]
            bf_ = b_ref[2 * l][0:1]
            bb_ = b_ref[2 * l + 1][0:1]
            xin_all = xs_in[...].reshape(T * P, 2 * H)
            gihf_s[...] = (jnp.dot(xin_all, wif, preferred_element_type=f32)
                           + bf_).reshape(T, P, 4 * H)
            gihb_s[...] = (jnp.dot(xin_all, wib, preferred_element_type=f32)
                           + bb_).reshape(T, P, 4 * H)

            def gih(d, t):
                return (gihf_s if d == 0 else gihb_s)[t]

        hf, cf, hb, cb = z, z, z, z
        accf, accb = z, z
        for s in range(T):
            tf, tb = s, T - 1 - s
            gf = gih(0, tf) + jnp.dot(hf.astype(bf16), whf,
                                      preferred_element_type=f32)
            gb = gih(1, tb) + jnp.dot(hb.astype(bf16), whb,
                                      preferred_element_type=f32)
            hf, cf = cell(gf, cf)
            hb, cb = cell(gb, cb)
            if l < L - 1:
                xs_out[tf, :, :H] = hf.astype(bf16)
                xs_out[tb, :, H:] = hb.astype(bf16)
            else:
                accf = accf + hf
                accb = accb + hb
        return accf, accb

    for l in range(L):
        xs_in, xs_out = (xsa, xsb) if l % 2 == 1 else (xsb, xsa)
        if l < L - 1:
            run_layer(l, xs_in, xs_out)
        else:
            accf, accb = run_layer(l, xs_in, xs_out)
            inv_t = f32(1.0 / T)
            out_ref[:, :H] = accf * inv_t
            out_ref[:, H:] = accb * inv_t


def _lstm_stage(xtp, m0, wh, wi, bb):
    grid = NRPAD // P
    return pl.pallas_call(
        _lstm_body,
        grid=(grid,),
        in_specs=[
            pl.BlockSpec((P, TPAD), lambda i: (i, 0)),
            pl.BlockSpec((2, TPAD, T * 4 * H), lambda i: (0, 0, 0)),
            pl.BlockSpec((2 * L, H, 4 * H), lambda i: (0, 0, 0)),
            pl.BlockSpec((2 * (L - 1), 2 * H, 4 * H), lambda i: (0, 0, 0)),
            pl.BlockSpec((2 * L, 8, 4 * H), lambda i: (0, 0, 0)),
        ],
        out_specs=pl.BlockSpec((P, 2 * H), lambda i: (i, 0)),
        out_shape=jax.ShapeDtypeStruct((NRPAD, 2 * H), jnp.float32),
        scratch_shapes=[
            pltpu.VMEM((T, P, 2 * H), jnp.bfloat16),
            pltpu.VMEM((T, P, 2 * H), jnp.bfloat16),
            pltpu.VMEM((T, P, 4 * H), jnp.float32),
            pltpu.VMEM((T, P, 4 * H), jnp.float32),
            pltpu.VMEM((P, T * 4 * H), jnp.float32),
            pltpu.VMEM((P, T * 4 * H), jnp.float32),
        ],
    )(xtp, m0, wh, wi, bb)


EPC = 128
NCHUNK = EPAD2 // EPC
NFLAT = NPAD * NPAD
ZCH = NFLAT // 16


def _adj_sc_body(edges_hbm, out_hbm, rows_v, cols_v, idx_v, ones_v,
                 zeros_v, c_sh):
    cid = jax.lax.axis_index("c")
    sid = jax.lax.axis_index("s")

    @pl.when((cid == 0) & (sid == 0))
    def _():
        pltpu.sync_copy(edges_hbm.at[0], rows_v)
        pltpu.sync_copy(edges_hbm.at[1], cols_v)

        def fill_ones(j, carry):
            ones_v[pl.ds(j * 16, 16)] = jnp.full((16,), 1.0, jnp.float32)
            return carry

        def fill_zeros(j, carry):
            zeros_v[pl.ds(j * 16, 16)] = jnp.zeros((16,), jnp.float32)
            return carry

        jax.lax.fori_loop(0, EPC // 16, fill_ones, 0)
        jax.lax.fori_loop(0, ZCH // 16, fill_zeros, 0)

        for j in range(NCHUNK):
            def flat_idx(k, carry, j=j):
                r = rows_v[pl.ds(j * EPC + k * 16, 16)]
                c = cols_v[pl.ds(j * EPC + k * 16, 16)]
                idx_v[j, pl.ds(k * 16, 16)] = c * NPAD + r
                return carry

            jax.lax.fori_loop(0, EPC // 16, flat_idx, 0)

        for k in range(16):
            pltpu.sync_copy(zeros_v, c_sh.at[pl.ds(k * ZCH, ZCH)])

        for j in range(NCHUNK):
            pltpu.sync_copy(ones_v, c_sh.at[idx_v.at[j]], add=True)

        pltpu.sync_copy(c_sh, out_hbm)


def _adj_stage(edges_p):
    mesh = plsc.VectorSubcoreMesh(core_axis_name="c", subcore_axis_name="s")
    return pl.kernel(
        _adj_sc_body,
        out_type=jax.ShapeDtypeStruct((NFLAT,), jnp.float32),
        mesh=mesh,
        scratch_types=[
            pltpu.VMEM((EPAD2,), jnp.int32),
            pltpu.VMEM((EPAD2,), jnp.int32),
            pltpu.VMEM((NCHUNK, EPC), jnp.int32),
            pltpu.VMEM((EPC,), jnp.float32),
            pltpu.VMEM((ZCH,), jnp.float32),
            pltpu.VMEM_SHARED((NFLAT,), jnp.float32),
        ],
    )(edges_p)


def _gcn_body(feats_ref, c_ref, gw_ref, gb_ref, nw_ref, nb_ref, na_ref,
              cw_ref, cb_ref, out_ref):
    f32 = jnp.float32
    rmask1 = (jax.lax.broadcasted_iota(jnp.int32, (NPAD, 1), 0) < N)
    rmaskf = rmask1.astype(f32)
    ii = jax.lax.broadcasted_iota(jnp.int32, (NPAD, NPAD), 0)
    jj = jax.lax.broadcasted_iota(jnp.int32, (NPAD, NPAD), 1)

    c = jnp.where((ii < N) & (jj < N), c_ref[...], f32(0.0))
    deg = jnp.sum(c, axis=1, keepdims=True) + 1.0
    dis = jax.lax.rsqrt(deg)
    eye = jnp.where((ii == jj) & (ii < N), f32(1.0), f32(0.0))
    eye_full = jnp.where(ii == jj, f32(1.0), f32(0.0))
    a_hat = jnp.dot((c + eye) * dis, eye_full * dis,
                    preferred_element_type=f32)

    inv_n = f32(1.0 / N)
    h = feats_ref[...]
    for j in range(3):
        hin = h
        w = gw_ref[j]
        bias = gb_ref[j][0:1]
        hw = jnp.dot(h.reshape(B * NPAD, GH), w,
                     preferred_element_type=f32).reshape(B, NPAD, GH)
        agg = jnp.stack(
            [jnp.dot(a_hat, hw[bb], preferred_element_type=f32)
             for bb in range(B)], axis=0)
        x = (agg + bias) * rmaskf
        mean = jnp.sum(x, axis=1) * inv_n
        xm = (x - na_ref[j][0:1] * mean[:, None, :]) * rmaskf
        var = jnp.sum(xm * xm, axis=1) * inv_n
        xm = xm / jnp.sqrt(var + 1e-5)[:, None, :]
        gn = xm * nw_ref[j][0:1] + nb_ref[j][0:1]
        h = jnp.maximum(gn * rmaskf + hin, 0.0)

    pooled = jnp.sum(h, axis=1) * inv_n
    logits = jnp.dot(pooled, cw_ref[...],
                     preferred_element_type=f32) + cb_ref[0, 0]
    out_ref[...] = jnp.broadcast_to(logits[:, 0:1], (B, 128))


def _gcn_stage(feats, c, gw, gb, nw, nb, na, cw, cb):
    return pl.pallas_call(
        _gcn_body,
        out_shape=jax.ShapeDtypeStruct((B, 128), jnp.float32),
    )(feats, c, gw, gb, nw, nb, na, cw, cb)


@jax.jit
def kernel(x, edge_index, params):
    f32 = jnp.float32
    p = params

    eye_tp = jnp.eye(TPAD, dtype=f32)[:, :T]
    ind_one = jnp.eye(TPAD, dtype=f32)[:, T]
    m0_list = []
    for d in ("fwd", "bwd"):
        w0v = p["W_ih_l0_%s" % d][:, 0]
        b0v = p["b_ih_l0_%s" % d] + p["b_hh_l0_%s" % d]
        m0d = (eye_tp[:, :, None] * w0v[None, None, :]
               + ind_one[:, None, None] * b0v[None, None, :])
        m0_list.append(m0d.reshape(TPAD, T * 4 * H))
    m0 = jnp.stack(m0_list, axis=0).astype(jnp.bfloat16)
    wh = jnp.stack([p["W_hh_l%d_%s" % (l, d)].T
                    for l in range(L) for d in ("fwd", "bwd")],
                   axis=0).astype(jnp.bfloat16)
    wi = jnp.stack([p["W_ih_l%d_%s" % (l, d)].T
                    for l in range(1, L) for d in ("fwd", "bwd")],
                   axis=0).astype(jnp.bfloat16)
    bb = jnp.zeros((2 * L, 8, 4 * H), f32)
    for l in range(L):
        for di, d in enumerate(("fwd", "bwd")):
            bb = bb.at[2 * l + di, 0].set(
                p["b_ih_l%d_%s" % (l, d)] + p["b_hh_l%d_%s" % (l, d)])

    xtp = jnp.transpose(x, (0, 2, 1)).reshape(NROWS, T)
    xtp = jnp.pad(xtp, ((0, NRPAD - NROWS), (0, TPAD - T)))
    xtp = xtp.at[:, T].set(1.0)

    edges_p = jnp.full((2, EPAD2), NPAD - 1, jnp.int32)
    edges_p = edges_p.at[:, :E].set(edge_index.astype(jnp.int32))
    c = _adj_stage(edges_p).reshape(NPAD, NPAD)

    node_feats = _lstm_stage(xtp, m0, wh, wi, bb)[:NROWS]
    feats = jnp.pad(node_feats.reshape(B, N, 2 * H),
                    ((0, 0), (0, NPAD - N), (0, 0)))

    gw = jnp.stack([p["gcn%d_W" % (j + 1)].T for j in range(3)], axis=0)
    gb = jnp.zeros((3, 8, GH), f32)
    nw = jnp.zeros((3, 8, GH), f32)
    nb = jnp.zeros((3, 8, GH), f32)
    na = jnp.zeros((3, 8, GH), f32)
    for j in range(3):
        gb = gb.at[j, 0].set(p["gcn%d_b" % (j + 1)])
        nw = nw.at[j, 0].set(p["norm%d_w" % (j + 1)])
        nb = nb.at[j, 0].set(p["norm%d_b" % (j + 1)])
        na = na.at[j, 0].set(p["norm%d_a" % (j + 1)])
    cw = jnp.zeros((GH, 128), f32).at[:, 0].set(p["cls_W"][0])
    cb = jnp.broadcast_to(p["cls_b"].reshape(1, 1), (8, 128))

    out = _gcn_stage(feats, c, gw, gb, nw, nb, na, cw, cb)
    return out[:, 0:1]


# speedup vs baseline: 1.0781x; 1.0781x over previous
import functools

import jax
import jax.numpy as jnp
import numpy as np
from jax.experimental import pallas as pl
from jax.experimental.pallas import tpu as pltpu
from jax.experimental.pallas import tpu_sc as plsc

H = 128
L = 3
GH = 256
B = 8
T = 12
N = 325
E = 2600

NPAD = 352
EPAD2 = 2688
TPAD = 16
P = 336
NROWS = B * N
NRPAD = 2688


def _lstm_body(x_ref, m0_ref, wh_ref, wi_ref, b_ref, out_ref, xsa, xsb,
               gihf_s, gihb_s, gih0f_s, gih0b_s):
    f32 = jnp.float32
    bf16 = jnp.bfloat16

    def sg(x):
        return 0.5 * jnp.tanh(0.5 * x) + 0.5

    def cell(g, c):
        ig = sg(g[:, :H])
        fg = sg(g[:, H:2 * H])
        gg = jnp.tanh(g[:, 2 * H:3 * H])
        og = sg(g[:, 3 * H:])
        c2 = fg * c + ig * gg
        h2 = og * jnp.tanh(c2)
        return h2, c2

    def dot_t(a, w):
        return jax.lax.dot_general(a, w, (((1,), (1,)), ((), ())),
                                   preferred_element_type=f32)

    def run_layer(l, xs_in, xs_out):
        whf = wh_ref[2 * l]
        whb = wh_ref[2 * l + 1]
        z = jnp.zeros((P, H), f32)

        if l == 0:
            xb = x_ref[...].astype(bf16)
            gih0f_s[...] = jnp.dot(xb, m0_ref[0], preferred_element_type=f32)
            gih0b_s[...] = jnp.dot(xb, m0_ref[1], preferred_element_type=f32)

            def gih(d, t):
                src = gih0f_s if d == 0 else gih0b_s
                return src[:, t * 4 * H:(t + 1) * 4 * H]
        else:
            wif = wi_ref[2 * (l - 1)]
            wib = wi_ref[2 * (l - 1) + 1]
            bf_ = b_ref[2 * (l - 1):2 * (l - 1) + 1]
            bb_ = b_ref[2 * (l - 1) + 1:2 * (l - 1) + 2]
            xin_all = xs_in[...].reshape(T * P, 2 * H)
            gihf_s[...] = (dot_t(xin_all, wif) + bf_).reshape(T, P, 4 * H)
            gihb_s[...] = (dot_t(xin_all, wib) + bb_).reshape(T, P, 4 * H)

            def gih(d, t):
                return (gihf_s if d == 0 else gihb_s)[t]

        hf, cf, hb, cb = z, z, z, z
        accf, accb = z, z
        for s in range(T):
            tf, tb = s, T - 1 - s
            gf = gih(0, tf) + dot_t(hf.astype(bf16), whf)
            gb = gih(1, tb) + dot_t(hb.astype(bf16), whb)
            hf, cf = cell(gf, cf)
            hb, cb = cell(gb, cb)
            if l < L - 1:
                xs_out[tf, :, :H] = hf.astype(bf16)
                xs_out[tb, :, H:] = hb.astype(bf16)
            else:
                accf = accf + hf
                accb = accb + hb
        return accf, accb

    for l in range(L):
        xs_in, xs_out = (xsa, xsb) if l % 2 == 1 else (xsb, xsa)
        if l < L - 1:
            run_layer(l, xs_in, xs_out)
        else:
            accf, accb = run_layer(l, xs_in, xs_out)
            inv_t = f32(1.0 / T)
            out_ref[:, :H] = accf * inv_t
            out_ref[:, H:] = accb * inv_t


def _lstm_stage(xtp, m0, wh, wi, bb):
    grid = NRPAD // P
    return pl.pallas_call(
        _lstm_body,
        grid=(grid,),
        in_specs=[
            pl.BlockSpec((P, TPAD), lambda i: (i, 0)),
            pl.BlockSpec((2, TPAD, T * 4 * H), lambda i: (0, 0, 0)),
            pl.BlockSpec((2 * L, 4 * H, H), lambda i: (0, 0, 0)),
            pl.BlockSpec((2 * (L - 1), 4 * H, 2 * H), lambda i: (0, 0, 0)),
            pl.BlockSpec((2 * (L - 1), 4 * H), lambda i: (0, 0)),
        ],
        out_specs=pl.BlockSpec((P, 2 * H), lambda i: (i, 0)),
        out_shape=jax.ShapeDtypeStruct((NRPAD, 2 * H), jnp.float32),
        scratch_shapes=[
            pltpu.VMEM((T, P, 2 * H), jnp.bfloat16),
            pltpu.VMEM((T, P, 2 * H), jnp.bfloat16),
            pltpu.VMEM((T, P, 4 * H), jnp.float32),
            pltpu.VMEM((T, P, 4 * H), jnp.float32),
            pltpu.VMEM((P, T * 4 * H), jnp.float32),
            pltpu.VMEM((P, T * 4 * H), jnp.float32),
        ],
    )(xtp, m0, wh, wi, bb)


EPC = 128
NCHUNK = EPAD2 // EPC
NFLAT = NPAD * NPAD
ZCH = NFLAT // 16


def _adj_sc_body(edges_hbm, out_hbm, rows_v, cols_v, idx_v, ones_v,
                 zeros_v, c_sh):
    cid = jax.lax.axis_index("c")
    sid = jax.lax.axis_index("s")

    @pl.when((cid == 0) & (sid == 0))
    def _():
        pltpu.sync_copy(edges_hbm.at[0], rows_v)
        pltpu.sync_copy(edges_hbm.at[1], cols_v)

        def fill_ones(j, carry):
            ones_v[pl.ds(j * 16, 16)] = jnp.full((16,), 1.0, jnp.float32)
            return carry

        def fill_zeros(j, carry):
            zeros_v[pl.ds(j * 16, 16)] = jnp.zeros((16,), jnp.float32)
            return carry

        jax.lax.fori_loop(0, EPC // 16, fill_ones, 0)
        jax.lax.fori_loop(0, ZCH // 16, fill_zeros, 0)

        for j in range(NCHUNK):
            def flat_idx(k, carry, j=j):
                r = rows_v[pl.ds(j * EPC + k * 16, 16)]
                c = cols_v[pl.ds(j * EPC + k * 16, 16)]
                idx_v[j, pl.ds(k * 16, 16)] = c * NPAD + r
                return carry

            jax.lax.fori_loop(0, EPC // 16, flat_idx, 0)

        for k in range(16):
            pltpu.sync_copy(zeros_v, c_sh.at[pl.ds(k * ZCH, ZCH)])

        for j in range(NCHUNK):
            pltpu.sync_copy(ones_v, c_sh.at[idx_v.at[j]], add=True)

        pltpu.sync_copy(c_sh, out_hbm)


def _adj_stage(edges_p):
    mesh = plsc.VectorSubcoreMesh(core_axis_name="c", subcore_axis_name="s")
    return pl.kernel(
        _adj_sc_body,
        out_type=jax.ShapeDtypeStruct((NFLAT,), jnp.float32),
        mesh=mesh,
        scratch_types=[
            pltpu.VMEM((EPAD2,), jnp.int32),
            pltpu.VMEM((EPAD2,), jnp.int32),
            pltpu.VMEM((NCHUNK, EPC), jnp.int32),
            pltpu.VMEM((EPC,), jnp.float32),
            pltpu.VMEM((ZCH,), jnp.float32),
            pltpu.VMEM_SHARED((NFLAT,), jnp.float32),
        ],
    )(edges_p)


def _gcn_body(nf_ref, c_ref, w1_ref, w2_ref, w3_ref, gb_ref, nw_ref, nb_ref,
              na_ref, cw_ref, out_ref):
    f32 = jnp.float32
    rmask1 = (jax.lax.broadcasted_iota(jnp.int32, (NPAD, 1), 0) < N)
    rmaskf = rmask1.astype(f32)
    ii = jax.lax.broadcasted_iota(jnp.int32, (NPAD, NPAD), 0)
    jj = jax.lax.broadcasted_iota(jnp.int32, (NPAD, NPAD), 1)

    c = jnp.where((ii < N) & (jj < N), c_ref[...], f32(0.0))
    deg = jnp.sum(c, axis=1, keepdims=True) + 1.0
    dis = jax.lax.rsqrt(deg)
    eye = jnp.where((ii == jj) & (ii < N), f32(1.0), f32(0.0))
    eye_full = jnp.where(ii == jj, f32(1.0), f32(0.0))
    a_hat = jnp.dot((c + eye) * dis, eye_full * dis,
                    preferred_element_type=f32)

    zpad = jnp.zeros((NPAD - N, GH), f32)
    h = jnp.stack([jnp.concatenate([nf_ref[bb * N:(bb + 1) * N], zpad], 0)
                   for bb in range(B)], axis=0)

    inv_n = f32(1.0 / N)
    for j, w_ref in enumerate((w1_ref, w2_ref, w3_ref)):
        hin = h
        hw = jax.lax.dot_general(
            h.reshape(B * NPAD, GH), w_ref[...], (((1,), (1,)), ((), ())),
            preferred_element_type=f32).reshape(B, NPAD, GH)
        agg = jnp.stack(
            [jnp.dot(a_hat, hw[bb], preferred_element_type=f32)
             for bb in range(B)], axis=0)
        x = (agg + gb_ref[j:j + 1]) * rmaskf
        mean = jnp.sum(x, axis=1) * inv_n
        xm = (x - na_ref[j:j + 1] * mean[:, None, :]) * rmaskf
        var = jnp.sum(xm * xm, axis=1) * inv_n
        xm = xm / jnp.sqrt(var + 1e-5)[:, None, :]
        gn = xm * nw_ref[j:j + 1] + nb_ref[j:j + 1]
        h = jnp.maximum(gn * rmaskf + hin, 0.0)

    pooled = jnp.sum(h, axis=1) * inv_n
    logits = jax.lax.dot_general(
        pooled, cw_ref[...], (((1,), (1,)), ((), ())),
        preferred_element_type=f32)
    out_ref[...] = jnp.broadcast_to(logits, (B, 128))


def _gcn_stage(nf, c, w1, w2, w3, gb, nw, nb, na, cw):
    return pl.pallas_call(
        _gcn_body,
        out_shape=jax.ShapeDtypeStruct((B, 128), jnp.float32),
    )(nf, c, w1, w2, w3, gb, nw, nb, na, cw)


@jax.jit
def kernel(x, edge_index, params):
    f32 = jnp.float32
    p = params

    eye_tp = jnp.eye(TPAD, dtype=f32)[:, :T]
    ind_one = jnp.eye(TPAD, dtype=f32)[:, T]
    m0_list = []
    for d in ("fwd", "bwd"):
        w0v = p["W_ih_l0_%s" % d][:, 0]
        b0v = p["b_ih_l0_%s" % d] + p["b_hh_l0_%s" % d]
        m0d = (eye_tp[:, :, None] * w0v[None, None, :]
               + ind_one[:, None, None] * b0v[None, None, :])
        m0_list.append(m0d.reshape(TPAD, T * 4 * H))
    m0 = jnp.stack(m0_list, axis=0).astype(jnp.bfloat16)
    wh = jnp.stack([p["W_hh_l%d_%s" % (l, d)]
                    for l in range(L) for d in ("fwd", "bwd")],
                   axis=0).astype(jnp.bfloat16)
    wi = jnp.stack([p["W_ih_l%d_%s" % (l, d)]
                    for l in range(1, L) for d in ("fwd", "bwd")],
                   axis=0).astype(jnp.bfloat16)
    bb = jnp.stack([p["b_ih_l%d_%s" % (l, d)] + p["b_hh_l%d_%s" % (l, d)]
                    for l in range(1, L) for d in ("fwd", "bwd")],
                   axis=0)

    xtp = jnp.transpose(x, (0, 2, 1)).reshape(NROWS, T)
    xtp = jnp.pad(xtp, ((0, NRPAD - NROWS), (0, TPAD - T)))
    xtp = xtp.at[:, T].set(1.0)

    edges_p = jnp.full((2, EPAD2), NPAD - 1, jnp.int32)
    edges_p = edges_p.at[:, :E].set(edge_index.astype(jnp.int32))
    c = _adj_stage(edges_p).reshape(NPAD, NPAD)

    node_feats = _lstm_stage(xtp, m0, wh, wi, bb)

    gb = jnp.stack([p["gcn%d_b" % (j + 1)] for j in range(3)], axis=0)
    nw = jnp.stack([p["norm%d_w" % (j + 1)] for j in range(3)], axis=0)
    nb = jnp.stack([p["norm%d_b" % (j + 1)] for j in range(3)], axis=0)
    na = jnp.stack([p["norm%d_a" % (j + 1)] for j in range(3)], axis=0)

    out = _gcn_stage(node_feats, c, p["gcn1_W"], p["gcn2_W"], p["gcn3_W"],
                     gb, nw, nb, na, p["cls_W"])
    return out[:, 0:1] + p["cls_b"]


# speedup vs baseline: 1.1691x; 1.0843x over previous
import functools

import jax
import jax.numpy as jnp
import numpy as np
from jax.experimental import pallas as pl
from jax.experimental.pallas import tpu as pltpu
from jax.experimental.pallas import tpu_sc as plsc

H = 128
L = 3
GH = 256
B = 8
T = 12
N = 325
E = 2600

NPAD = 352
EPAD2 = 2688
TPAD = 16
P = 672
NROWS = B * N
NRPAD = 2688


def _lstm_body(x_ref, w0_ref, wh_ref, wi_ref, b_ref, out_ref, xsa, xsb,
               gihf_s, gihb_s):
    f32 = jnp.float32
    bf16 = jnp.bfloat16

    def sg(x):
        return 0.5 * jnp.tanh(0.5 * x) + 0.5

    def cell(g, c):
        ig = sg(g[:, :H])
        fg = sg(g[:, H:2 * H])
        gg = jnp.tanh(g[:, 2 * H:3 * H])
        og = sg(g[:, 3 * H:])
        c2 = fg * c + ig * gg
        h2 = og * jnp.tanh(c2)
        return h2, c2

    def dot_t(a, w):
        return jax.lax.dot_general(a, w, (((1,), (1,)), ((), ())),
                                   preferred_element_type=f32)

    def run_layer(l, xs_in, xs_out):
        whf = wh_ref[2 * l]
        whb = wh_ref[2 * l + 1]
        z = jnp.zeros((P, H), f32)

        if l == 0:
            w0f = w0_ref[0:1]
            w0b = w0_ref[1:2]
            b0f = b_ref[0:1]
            b0b = b_ref[1:2]

            def gih(d, t):
                if d == 0:
                    return x_ref[:, t:t + 1] * w0f + b0f
                return x_ref[:, t:t + 1] * w0b + b0b
        else:
            wif = wi_ref[2 * (l - 1)]
            wib = wi_ref[2 * (l - 1) + 1]
            bf_ = b_ref[2 * l:2 * l + 1]
            bb_ = b_ref[2 * l + 1:2 * l + 2]
            xin_all = xs_in[...].reshape(T * P, 2 * H)
            gihf_s[...] = (dot_t(xin_all, wif) + bf_).reshape(T, P, 4 * H)
            gihb_s[...] = (dot_t(xin_all, wib) + bb_).reshape(T, P, 4 * H)

            def gih(d, t):
                return (gihf_s if d == 0 else gihb_s)[t]

        hf, cf, hb, cb = z, z, z, z
        accf, accb = z, z
        for s in range(T):
            tf, tb = s, T - 1 - s
            gf = gih(0, tf) + dot_t(hf.astype(bf16), whf)
            gb = gih(1, tb) + dot_t(hb.astype(bf16), whb)
            hf, cf = cell(gf, cf)
            hb, cb = cell(gb, cb)
            if l < L - 1:
                xs_out[tf, :, :H] = hf.astype(bf16)
                xs_out[tb, :, H:] = hb.astype(bf16)
            else:
                accf = accf + hf
                accb = accb + hb
        return accf, accb

    for l in range(L):
        xs_in, xs_out = (xsa, xsb) if l % 2 == 1 else (xsb, xsa)
        if l < L - 1:
            run_layer(l, xs_in, xs_out)
        else:
            accf, accb = run_layer(l, xs_in, xs_out)
            inv_t = f32(1.0 / T)
            out_ref[:, :H] = accf * inv_t
            out_ref[:, H:] = accb * inv_t


def _lstm_stage(xtp, w0, wh, wi, bb):
    grid = NRPAD // P
    return pl.pallas_call(
        _lstm_body,
        grid=(grid,),
        in_specs=[
            pl.BlockSpec((P, TPAD), lambda i: (i, 0)),
            pl.BlockSpec((2, 4 * H), lambda i: (0, 0)),
            pl.BlockSpec((2 * L, 4 * H, H), lambda i: (0, 0, 0)),
            pl.BlockSpec((2 * (L - 1), 4 * H, 2 * H), lambda i: (0, 0, 0)),
            pl.BlockSpec((2 * L, 4 * H), lambda i: (0, 0)),
        ],
        out_specs=pl.BlockSpec((P, 2 * H), lambda i: (i, 0)),
        out_shape=jax.ShapeDtypeStruct((NRPAD, 2 * H), jnp.float32),
        scratch_shapes=[
            pltpu.VMEM((T, P, 2 * H), jnp.bfloat16),
            pltpu.VMEM((T, P, 2 * H), jnp.bfloat16),
            pltpu.VMEM((T, P, 4 * H), jnp.float32),
            pltpu.VMEM((T, P, 4 * H), jnp.float32),
        ],
    )(xtp, w0, wh, wi, bb)


EPC = 128
NCHUNK = EPAD2 // EPC
NFLAT = NPAD * NPAD
ZCH = NFLAT // 16


def _adj_sc_body(edges_hbm, out_hbm, rows_v, cols_v, idx_v, ones_v,
                 zeros_v, c_sh):
    cid = jax.lax.axis_index("c")
    sid = jax.lax.axis_index("s")

    @pl.when((cid == 0) & (sid == 0))
    def _():
        pltpu.sync_copy(edges_hbm.at[0], rows_v)
        pltpu.sync_copy(edges_hbm.at[1], cols_v)

        def fill_ones(j, carry):
            ones_v[pl.ds(j * 16, 16)] = jnp.full((16,), 1.0, jnp.float32)
            return carry

        def fill_zeros(j, carry):
            zeros_v[pl.ds(j * 16, 16)] = jnp.zeros((16,), jnp.float32)
            return carry

        jax.lax.fori_loop(0, EPC // 16, fill_ones, 0)
        jax.lax.fori_loop(0, ZCH // 16, fill_zeros, 0)

        for j in range(NCHUNK):
            def flat_idx(k, carry, j=j):
                r = rows_v[pl.ds(j * EPC + k * 16, 16)]
                c = cols_v[pl.ds(j * EPC + k * 16, 16)]
                idx_v[j, pl.ds(k * 16, 16)] = c * NPAD + r
                return carry

            jax.lax.fori_loop(0, EPC // 16, flat_idx, 0)

        for k in range(16):
            pltpu.sync_copy(zeros_v, c_sh.at[pl.ds(k * ZCH, ZCH)])

        for j in range(NCHUNK):
            pltpu.sync_copy(ones_v, c_sh.at[idx_v.at[j]], add=True)

        pltpu.sync_copy(c_sh, out_hbm)


def _adj_stage(edges_p):
    mesh = plsc.VectorSubcoreMesh(core_axis_name="c", subcore_axis_name="s")
    return pl.kernel(
        _adj_sc_body,
        out_type=jax.ShapeDtypeStruct((NFLAT,), jnp.float32),
        mesh=mesh,
        scratch_types=[
            pltpu.VMEM((EPAD2,), jnp.int32),
            pltpu.VMEM((EPAD2,), jnp.int32),
            pltpu.VMEM((NCHUNK, EPC), jnp.int32),
            pltpu.VMEM((EPC,), jnp.float32),
            pltpu.VMEM((ZCH,), jnp.float32),
            pltpu.VMEM_SHARED((NFLAT,), jnp.float32),
        ],
    )(edges_p)


def _gcn_body(nf_ref, c_ref, w1_ref, w2_ref, w3_ref, gb_ref, nw_ref, nb_ref,
              na_ref, cw_ref, out_ref):
    f32 = jnp.float32
    rmask1 = (jax.lax.broadcasted_iota(jnp.int32, (NPAD, 1), 0) < N)
    rmaskf = rmask1.astype(f32)
    ii = jax.lax.broadcasted_iota(jnp.int32, (NPAD, NPAD), 0)
    jj = jax.lax.broadcasted_iota(jnp.int32, (NPAD, NPAD), 1)

    c = jnp.where((ii < N) & (jj < N), c_ref[...], f32(0.0))
    deg = jnp.sum(c, axis=1, keepdims=True) + 1.0
    dis = jax.lax.rsqrt(deg)
    eye = jnp.where((ii == jj) & (ii < N), f32(1.0), f32(0.0))
    eye_full = jnp.where(ii == jj, f32(1.0), f32(0.0))
    a_hat = jnp.dot((c + eye) * dis, eye_full * dis,
                    preferred_element_type=f32)

    zpad = jnp.zeros((NPAD - N, GH), f32)
    h = jnp.stack([jnp.concatenate([nf_ref[bb * N:(bb + 1) * N], zpad], 0)
                   for bb in range(B)], axis=0)

    inv_n = f32(1.0 / N)
    for j, w_ref in enumerate((w1_ref, w2_ref, w3_ref)):
        hin = h
        hw = jax.lax.dot_general(
            h.reshape(B * NPAD, GH), w_ref[...], (((1,), (1,)), ((), ())),
            preferred_element_type=f32).reshape(B, NPAD, GH)
        agg = jnp.stack(
            [jnp.dot(a_hat, hw[bb], preferred_element_type=f32)
             for bb in range(B)], axis=0)
        x = (agg + gb_ref[j:j + 1]) * rmaskf
        mean = jnp.sum(x, axis=1) * inv_n
        xm = (x - na_ref[j:j + 1] * mean[:, None, :]) * rmaskf
        var = jnp.sum(xm * xm, axis=1) * inv_n
        xm = xm / jnp.sqrt(var + 1e-5)[:, None, :]
        gn = xm * nw_ref[j:j + 1] + nb_ref[j:j + 1]
        h = jnp.maximum(gn * rmaskf + hin, 0.0)

    pooled = jnp.sum(h, axis=1) * inv_n
    logits = jax.lax.dot_general(
        pooled, cw_ref[...], (((1,), (1,)), ((), ())),
        preferred_element_type=f32)
    out_ref[...] = jnp.broadcast_to(logits, (B, 128))


def _gcn_stage(nf, c, w1, w2, w3, gb, nw, nb, na, cw):
    return pl.pallas_call(
        _gcn_body,
        out_shape=jax.ShapeDtypeStruct((B, 128), jnp.float32),
    )(nf, c, w1, w2, w3, gb, nw, nb, na, cw)


@jax.jit
def kernel(x, edge_index, params):
    f32 = jnp.float32
    p = params

    w0 = jnp.stack([p["W_ih_l0_fwd"][:, 0], p["W_ih_l0_bwd"][:, 0]],
                   axis=0)
    wh = jnp.stack([p["W_hh_l%d_%s" % (l, d)]
                    for l in range(L) for d in ("fwd", "bwd")],
                   axis=0).astype(jnp.bfloat16)
    wi = jnp.stack([p["W_ih_l%d_%s" % (l, d)]
                    for l in range(1, L) for d in ("fwd", "bwd")],
                   axis=0).astype(jnp.bfloat16)
    bb = jnp.stack([p["b_ih_l%d_%s" % (l, d)] + p["b_hh_l%d_%s" % (l, d)]
                    for l in range(L) for d in ("fwd", "bwd")],
                   axis=0)

    xtp = jnp.transpose(x, (0, 2, 1)).reshape(NROWS, T)
    xtp = jnp.pad(xtp, ((0, NRPAD - NROWS), (0, TPAD - T)))

    edges_p = jnp.full((2, EPAD2), NPAD - 1, jnp.int32)
    edges_p = edges_p.at[:, :E].set(edge_index.astype(jnp.int32))
    c = _adj_stage(edges_p).reshape(NPAD, NPAD)

    node_feats = _lstm_stage(xtp, w0, wh, wi, bb)

    gb = jnp.stack([p["gcn%d_b" % (j + 1)] for j in range(3)], axis=0)
    nw = jnp.stack([p["norm%d_w" % (j + 1)] for j in range(3)], axis=0)
    nb = jnp.stack([p["norm%d_b" % (j + 1)] for j in range(3)], axis=0)
    na = jnp.stack([p["norm%d_a" % (j + 1)] for j in range(3)], axis=0)

    out = _gcn_stage(node_feats, c, p["gcn1_W"], p["gcn2_W"], p["gcn3_W"],
                     gb, nw, nb, na, p["cls_W"])
    return out[:, 0:1] + p["cls_b"]
